# baseline (speedup 1.0000x reference)
import functools

import jax
import jax.numpy as jnp
from jax import lax
from jax.experimental import pallas as pl
from jax.experimental.pallas import tpu as pltpu
from jax.experimental.pallas import tpu_sc as plsc

DIM = 16
NN, NE, NE3 = 10000, 160000, 50000
NW, CHUNK = 32, 128
JG = 40
PAD_E = NW * JG * CHUNK
EPW = JG * CHUNK
NPAD = 10240
NSUB = 16
RS = NPAD // NSUB
N3PAD = 57344
JG3 = 2 * N3PAD // (NW * CHUNK)
BE = 2048
BR = 4096


def _k0_body(x_ref, wnt_ref, bn_ref, o_ref):
    o_ref[...] = jnp.maximum(
        jnp.dot(x_ref[...], wnt_ref[...], preferred_element_type=jnp.float32)
        + bn_ref[...], 0.0)


def _k1_body(attr_ref, wet_ref, be_ref, w1t_ref, b1_ref, w2t_ref, b2_ref,
             we_ref):
    ea = jnp.maximum(
        jnp.dot(attr_ref[...], wet_ref[...],
                preferred_element_type=jnp.float32) + be_ref[...], 0.0)
    hh = jnp.maximum(
        jnp.dot(ea, w1t_ref[...], preferred_element_type=jnp.float32)
        + b1_ref[...], 0.0)
    we = (jnp.dot(hh, w2t_ref[...], preferred_element_type=jnp.float32)
          + b2_ref[...])
    we_ref[...] = we.astype(jnp.bfloat16)


def _km_body(xj_ref, we_ref, rt_ref, m_ref):
    xr = jnp.dot(xj_ref[...], rt_ref[...], preferred_element_type=jnp.float32)
    s = xr * we_ref[...].astype(jnp.float32)
    for half in (128, 64, 32, 16):
        s = s[:, :half] + s[:, half:2 * half]
    m_ref[...] = s


def _kgru_body(p_ref, q_ref, cnt_ref, h_ref, cb_ref, wih_ref, bih_ref,
               whh_ref, bhh_ref, o_ref):
    cnt = cnt_ref[0] + cnt_ref[1]
    denom = jnp.maximum(cnt, 1.0)
    agg = (p_ref[0] + p_ref[1] + q_ref[0] + q_ref[1]) / denom
    m2 = jnp.maximum(agg + cb_ref[...], 0.0)
    h = h_ref[...]
    gi = (jnp.dot(m2, wih_ref[...], preferred_element_type=jnp.float32)
          + bih_ref[...])
    gh = (jnp.dot(h, whh_ref[...], preferred_element_type=jnp.float32)
          + bhh_ref[...])
    r = jax.nn.sigmoid(gi[:, :16] + gh[:, :16])
    z = jax.nn.sigmoid(gi[:, 16:32] + gh[:, 16:32])
    n = jnp.tanh(gi[:, 32:48] + r * gh[:, 32:48])
    o_ref[...] = (1.0 - z) * n + z * h


def _kr1_body(t0_ref, t1_ref, acc_ref):
    i = pl.program_id(0)
    t0 = t0_ref[...]
    t1 = t1_ref[...]
    yhat = jnp.concatenate([(t0 + t1) * 0.5, t0 * t1, (t0 - t1) ** 2], axis=1)
    rows = lax.broadcasted_iota(jnp.int32, (BR, 1), 0) + i * BR
    maskf = jnp.where(rows < NE3, 1.0, 0.0)
    ym = yhat * maskf
    st = jnp.concatenate([jnp.sum(ym, axis=0)[None, :],
                          jnp.sum(yhat * ym, axis=0)[None, :]], axis=0)

    @pl.when(i == 0)
    def _():
        acc_ref[...] = st

    @pl.when(i > 0)
    def _():
        acc_ref[...] += st


def _kr2_body(t0_ref, t1_ref, ea3_ref, st_ref, wwt_ref, wbt_ref, g_ref, b_ref,
              o_ref):
    st = st_ref[...]
    mu = st[0:1, :] * (1.0 / NE3)
    var = st[1:2, :] * (1.0 / NE3) - mu * mu
    a = g_ref[...] * lax.rsqrt(var + 1e-5)
    c = b_ref[...] - mu * a
    t0 = t0_ref[...]
    t1 = t1_ref[...]
    yhat = jnp.concatenate([(t0 + t1) * 0.5, t0 * t1, (t0 - t1) ** 2], axis=1)
    ea3 = ea3_ref[...]
    w = jnp.dot(ea3, wwt_ref[...], preferred_element_type=jnp.float32)
    bias = jnp.dot(ea3, wbt_ref[...], preferred_element_type=jnp.float32)
    o_ref[...] = jnp.sum((yhat * a + c) * w, axis=1, keepdims=True) + bias


def _sc_mesh():
    return plsc.VectorSubcoreMesh(core_axis_name="c", subcore_axis_name="s")


_SC_PARAMS = pltpu.CompilerParams(use_tc_tiling_on_sc=False)


def _sc_gather(table, idx3d, n_chunks, out_rows, group):
    epw = n_chunks * CHUNK

    @functools.partial(
        pl.kernel,
        out_type=jax.ShapeDtypeStruct((out_rows, DIM), jnp.float32),
        mesh=_sc_mesh(),
        scratch_types=[pltpu.VMEM((n_chunks, CHUNK), jnp.int32),
                       pltpu.VMEM((epw, DIM), jnp.float32),
                       pltpu.SemaphoreType.DMA],
        compiler_params=_SC_PARAMS,
    )
    def kern(tab_hbm, idx_hbm, out_hbm, idx_v, buf_v, sem):
        cid = lax.axis_index("c")
        sid = lax.axis_index("s")
        wid = sid * 2 + cid
        pltpu.sync_copy(idx_hbm.at[wid], idx_v)

        @pl.loop(0, n_chunks, step=group)
        def _(g):
            descs = [
                pltpu.async_copy(tab_hbm.at[idx_v.at[g + u]],
                                 buf_v.at[pl.ds((g + u) * CHUNK, CHUNK)], sem)
                for u in range(group)
            ]
            for d in descs:
                d.wait()

        pltpu.sync_copy(buf_v, out_hbm.at[pl.ds(wid * epw, epw)])

    return kern(table, idx3d)


def _sc_scatter_add(m, dst3d, zeros_nd, n_chunks):
    epw = n_chunks * CHUNK

    @functools.partial(
        pl.kernel,
        out_type=jax.ShapeDtypeStruct((2, NPAD, DIM), jnp.float32),
        mesh=_sc_mesh(),
        scratch_types=[pltpu.VMEM((n_chunks, CHUNK), jnp.int32),
                       pltpu.VMEM((epw, DIM), jnp.float32),
                       pltpu.VMEM_SHARED((NPAD, DIM), jnp.float32),
                       pltpu.SemaphoreType.DMA],
        compiler_params=_SC_PARAMS,
    )
    def kern(m_hbm, idx_hbm, z_hbm, out_hbm, idx_v, m_v, acc, sem):
        cid = lax.axis_index("c")
        sid = lax.axis_index("s")
        wid = sid * 2 + cid
        slab = pl.ds(sid * RS, RS)
        pltpu.sync_copy(z_hbm.at[slab], acc.at[slab])
        d1 = pltpu.async_copy(idx_hbm.at[wid], idx_v, sem)
        d2 = pltpu.async_copy(m_hbm.at[pl.ds(wid * epw, epw)], m_v, sem)
        d1.wait()
        d2.wait()
        plsc.subcore_barrier()

        @pl.loop(0, n_chunks, step=4)
        def _(g):
            descs = [
                pltpu.async_copy(m_v.at[pl.ds((g + u) * CHUNK, CHUNK)],
                                 acc.at[idx_v.at[g + u]], sem, add=True)
                for u in range(4)
            ]
            for d in descs:
                d.wait()

        plsc.subcore_barrier()
        pltpu.sync_copy(acc.at[slab], out_hbm.at[cid, slab])

    return kern(m, dst3d, zeros_nd)


def _sc_count(dst3d, ones_cd, zeros_nd):
    @functools.partial(
        pl.kernel,
        out_type=jax.ShapeDtypeStruct((2, NPAD, DIM), jnp.float32),
        mesh=_sc_mesh(),
        scratch_types=[pltpu.VMEM((JG, CHUNK), jnp.int32),
                       pltpu.VMEM((CHUNK, DIM), jnp.float32),
                       pltpu.VMEM_SHARED((NPAD, DIM), jnp.float32),
                       pltpu.SemaphoreType.DMA],
        compiler_params=_SC_PARAMS,
    )
    def kern(idx_hbm, ones_hbm, z_hbm, out_hbm, idx_v, ones_v, acc, sem):
        cid = lax.axis_index("c")
        sid = lax.axis_index("s")
        wid = sid * 2 + cid
        slab = pl.ds(sid * RS, RS)
        pltpu.sync_copy(z_hbm.at[slab], acc.at[slab])
        pltpu.sync_copy(ones_hbm, ones_v)
        pltpu.sync_copy(idx_hbm.at[wid], idx_v)
        plsc.subcore_barrier()

        @pl.loop(0, JG, step=4)
        def _(g):
            descs = [
                pltpu.async_copy(ones_v, acc.at[idx_v.at[g + u]], sem,
                                 add=True)
                for u in range(4)
            ]
            for d in descs:
                d.wait()

        plsc.subcore_barrier()
        pltpu.sync_copy(acc.at[slab], out_hbm.at[cid, slab])

    return kern(dst3d, ones_cd, zeros_nd)


def kernel(x, edge_attr, edge_attr3, Wn, bn, We, be, W1, b1, W2, b2, conv_bias,
           w_ih, w_hh, b_ih, b_hh, Ww, Wb, gamma, beta, edge_index,
           edge_index3):
    f32 = jnp.float32
    xp = jnp.zeros((NPAD, 8), f32).at[:NN].set(x)
    attr = jnp.zeros((PAD_E, 19), f32).at[:NE].set(edge_attr)
    srcf = jnp.zeros((PAD_E,), jnp.int32).at[:NE].set(edge_index[0])
    dstf = jnp.full((PAD_E,), NN, jnp.int32).at[:NE].set(edge_index[1])
    HALF = PAD_E // 2
    JGH = JG // 2
    src_h = [srcf[:HALF].reshape(NW, JGH, CHUNK),
             srcf[HALF:].reshape(NW, JGH, CHUNK)]
    dst_h = [dstf[:HALF].reshape(NW, JGH, CHUNK),
             dstf[HALF:].reshape(NW, JGH, CHUNK)]
    dst3d = dstf.reshape(NW, JG, CHUNK)
    s3 = jnp.zeros((N3PAD,), jnp.int32).at[:NE3].set(edge_index3[0])
    d3 = jnp.zeros((N3PAD,), jnp.int32).at[:NE3].set(edge_index3[1])
    idx3 = jnp.concatenate([s3, d3]).reshape(NW, JG3, CHUNK)
    ea3 = jnp.zeros((N3PAD, 8), f32).at[:NE3].set(edge_attr3)
    zeros_nd = jnp.zeros((NPAD, DIM), f32)
    ones_cd = jnp.ones((CHUNK, DIM), f32)
    rt = jnp.repeat(jnp.eye(DIM, dtype=f32), DIM, axis=1)

    wnt = Wn.T
    wet = We.T
    w1t = W1.T
    w2t = W2.T
    wiht = w_ih.T
    whht = w_hh.T
    wwt = Ww.T
    wbt = Wb.T
    bn2 = bn.reshape(1, DIM)
    be2 = be.reshape(1, 12)
    b12 = b1.reshape(1, 128)
    b22 = b2.reshape(1, 256)
    cb2 = conv_bias.reshape(1, DIM)
    bih2 = b_ih.reshape(1, 48)
    bhh2 = b_hh.reshape(1, 48)
    g2 = gamma.reshape(1, 48)
    bt2 = beta.reshape(1, 48)

    out0 = pl.pallas_call(
        _k0_body,
        out_shape=jax.ShapeDtypeStruct((NPAD, DIM), f32),
    )(xp, wnt, bn2)

    n_eb = PAD_E // BE
    we = pl.pallas_call(
        _k1_body,
        grid=(n_eb,),
        in_specs=[pl.BlockSpec((BE, 19), lambda i: (i, 0)),
                  pl.BlockSpec((19, 12), lambda i: (0, 0)),
                  pl.BlockSpec((1, 12), lambda i: (0, 0)),
                  pl.BlockSpec((12, 128), lambda i: (0, 0)),
                  pl.BlockSpec((1, 128), lambda i: (0, 0)),
                  pl.BlockSpec((128, 256), lambda i: (0, 0)),
                  pl.BlockSpec((1, 256), lambda i: (0, 0))],
        out_specs=pl.BlockSpec((BE, 256), lambda i: (i, 0)),
        out_shape=jax.ShapeDtypeStruct((PAD_E, 256), jnp.bfloat16),
    )(attr, wet, be2, w1t, b12, w2t, b22)

    cnt = _sc_count(dst3d, ones_cd, zeros_nd)

    n_ebh = HALF // BE

    def km_half(xj, hf):
        return pl.pallas_call(
            _km_body,
            grid=(n_ebh,),
            in_specs=[pl.BlockSpec((BE, DIM), lambda i: (i, 0)),
                      pl.BlockSpec((BE, 256),
                                   lambda i, hf=hf: (i + hf * n_ebh, 0)),
                      pl.BlockSpec((DIM, 256), lambda i: (0, 0))],
            out_specs=pl.BlockSpec((BE, DIM), lambda i: (i, 0)),
            out_shape=jax.ShapeDtypeStruct((HALF, DIM), f32),
        )(xj, we, rt)

    h = out0
    for _ in range(3):
        xj0 = _sc_gather(h, src_h[0], JGH, HALF, 10)
        m0 = km_half(xj0, 0)
        xj1 = _sc_gather(h, src_h[1], JGH, HALF, 10)
        m1 = km_half(xj1, 1)
        part0 = _sc_scatter_add(m0, dst_h[0], zeros_nd, JGH)
        part1 = _sc_scatter_add(m1, dst_h[1], zeros_nd, JGH)
        h = pl.pallas_call(
            _kgru_body,
            out_shape=jax.ShapeDtypeStruct((NPAD, DIM), f32),
        )(part0, part1, cnt, h, cb2, wiht, bih2, whht, bhh2)

    temp = _sc_gather(h, idx3, JG3, 2 * N3PAD, 7)

    n_rb = N3PAD // BR
    t0spec = pl.BlockSpec((BR, DIM), lambda i: (i, 0))
    t1spec = pl.BlockSpec((BR, DIM), lambda i: (i + n_rb, 0))
    stats = pl.pallas_call(
        _kr1_body,
        grid=(n_rb,),
        in_specs=[t0spec, t1spec],
        out_specs=pl.BlockSpec((2, 48), lambda i: (0, 0)),
        out_shape=jax.ShapeDtypeStruct((2, 48), f32),
    )(temp, temp)
    res = pl.pallas_call(
        _kr2_body,
        grid=(n_rb,),
        in_specs=[t0spec, t1spec,
                  pl.BlockSpec((BR, 8), lambda i: (i, 0)),
                  pl.BlockSpec((2, 48), lambda i: (0, 0)),
                  pl.BlockSpec((8, 48), lambda i: (0, 0)),
                  pl.BlockSpec((8, 1), lambda i: (0, 0)),
                  pl.BlockSpec((1, 48), lambda i: (0, 0)),
                  pl.BlockSpec((1, 48), lambda i: (0, 0))],
        out_specs=pl.BlockSpec((BR, 1), lambda i: (i, 0)),
        out_shape=jax.ShapeDtypeStruct((N3PAD, 1), f32),
    )(temp, temp, ea3, stats, wwt, wbt, g2, bt2)
    return res[:NE3, 0]


# speedup vs baseline: 1.0310x; 1.0310x over previous
import functools

import jax
import jax.numpy as jnp
from jax import lax
from jax.experimental import pallas as pl
from jax.experimental.pallas import tpu as pltpu
from jax.experimental.pallas import tpu_sc as plsc

DIM = 16
NN, NE, NE3 = 10000, 160000, 50000
NW, CHUNK = 32, 128
JG = 40
PAD_E = NW * JG * CHUNK
EPW = JG * CHUNK
NPAD = 10240
NSUB = 16
RS = NPAD // NSUB
N3PAD = 57344
JG3 = 2 * N3PAD // (NW * CHUNK)
BE = 2048
BR = 4096


def _k0_body(x_ref, wnt_ref, bn_ref, o_ref):
    o_ref[...] = jnp.maximum(
        jnp.dot(x_ref[...], wnt_ref[...], preferred_element_type=jnp.float32)
        + bn_ref[...], 0.0)


def _k1_body(attr_ref, wet_ref, be_ref, w1t_ref, b1_ref, hh_ref):
    ea = jnp.maximum(
        jnp.dot(attr_ref[...], wet_ref[...],
                preferred_element_type=jnp.float32) + be_ref[...], 0.0)
    hh = jnp.maximum(
        jnp.dot(ea, w1t_ref[...], preferred_element_type=jnp.float32)
        + b1_ref[...], 0.0)
    hh_ref[...] = hh.astype(jnp.bfloat16)


def _km_body(xj_ref, hh_ref, w2t_ref, b2_ref, rt_ref, m_ref):
    we = (jnp.dot(hh_ref[...], w2t_ref[...],
                  preferred_element_type=jnp.float32) + b2_ref[...])
    xr = jnp.dot(xj_ref[...], rt_ref[...], preferred_element_type=jnp.float32)
    s = xr * we
    for half in (128, 64, 32, 16):
        s = s[:, :half] + s[:, half:2 * half]
    m_ref[...] = s


def _kgru_body(p_ref, q_ref, cnt_ref, h_ref, cb_ref, wih_ref, bih_ref,
               whh_ref, bhh_ref, o_ref):
    cnt = cnt_ref[0] + cnt_ref[1]
    denom = jnp.maximum(cnt, 1.0)
    agg = (p_ref[0] + p_ref[1] + q_ref[0] + q_ref[1]) / denom
    m2 = jnp.maximum(agg + cb_ref[...], 0.0)
    h = h_ref[...]
    gi = (jnp.dot(m2, wih_ref[...], preferred_element_type=jnp.float32)
          + bih_ref[...])
    gh = (jnp.dot(h, whh_ref[...], preferred_element_type=jnp.float32)
          + bhh_ref[...])
    r = jax.nn.sigmoid(gi[:, :16] + gh[:, :16])
    z = jax.nn.sigmoid(gi[:, 16:32] + gh[:, 16:32])
    n = jnp.tanh(gi[:, 32:48] + r * gh[:, 32:48])
    o_ref[...] = (1.0 - z) * n + z * h


def _kr1_body(t0_ref, t1_ref, acc_ref):
    i = pl.program_id(0)
    t0 = t0_ref[...]
    t1 = t1_ref[...]
    yhat = jnp.concatenate([(t0 + t1) * 0.5, t0 * t1, (t0 - t1) ** 2], axis=1)
    rows = lax.broadcasted_iota(jnp.int32, (BR, 1), 0) + i * BR
    maskf = jnp.where(rows < NE3, 1.0, 0.0)
    ym = yhat * maskf
    st = jnp.concatenate([jnp.sum(ym, axis=0)[None, :],
                          jnp.sum(yhat * ym, axis=0)[None, :]], axis=0)

    @pl.when(i == 0)
    def _():
        acc_ref[...] = st

    @pl.when(i > 0)
    def _():
        acc_ref[...] += st


def _kr2_body(t0_ref, t1_ref, ea3_ref, st_ref, wwt_ref, wbt_ref, g_ref, b_ref,
              o_ref):
    st = st_ref[...]
    mu = st[0:1, :] * (1.0 / NE3)
    var = st[1:2, :] * (1.0 / NE3) - mu * mu
    a = g_ref[...] * lax.rsqrt(var + 1e-5)
    c = b_ref[...] - mu * a
    t0 = t0_ref[...]
    t1 = t1_ref[...]
    yhat = jnp.concatenate([(t0 + t1) * 0.5, t0 * t1, (t0 - t1) ** 2], axis=1)
    ea3 = ea3_ref[...]
    w = jnp.dot(ea3, wwt_ref[...], preferred_element_type=jnp.float32)
    bias = jnp.dot(ea3, wbt_ref[...], preferred_element_type=jnp.float32)
    o_ref[...] = jnp.sum((yhat * a + c) * w, axis=1, keepdims=True) + bias


def _sc_mesh():
    return plsc.VectorSubcoreMesh(core_axis_name="c", subcore_axis_name="s")


_SC_PARAMS = pltpu.CompilerParams(use_tc_tiling_on_sc=False)


def _sc_gather(table, idx3d, n_chunks, out_rows, group):
    epw = n_chunks * CHUNK

    @functools.partial(
        pl.kernel,
        out_type=jax.ShapeDtypeStruct((out_rows, DIM), jnp.float32),
        mesh=_sc_mesh(),
        scratch_types=[pltpu.VMEM((n_chunks, CHUNK), jnp.int32),
                       pltpu.VMEM((epw, DIM), jnp.float32),
                       pltpu.SemaphoreType.DMA],
        compiler_params=_SC_PARAMS,
    )
    def kern(tab_hbm, idx_hbm, out_hbm, idx_v, buf_v, sem):
        cid = lax.axis_index("c")
        sid = lax.axis_index("s")
        wid = sid * 2 + cid
        pltpu.sync_copy(idx_hbm.at[wid], idx_v)

        @pl.loop(0, n_chunks, step=group)
        def _(g):
            descs = [
                pltpu.async_copy(tab_hbm.at[idx_v.at[g + u]],
                                 buf_v.at[pl.ds((g + u) * CHUNK, CHUNK)], sem)
                for u in range(group)
            ]
            for d in descs:
                d.wait()

        pltpu.sync_copy(buf_v, out_hbm.at[pl.ds(wid * epw, epw)])

    return kern(table, idx3d)


def _sc_scatter_add(m, dst3d, zeros_nd, n_chunks):
    epw = n_chunks * CHUNK

    @functools.partial(
        pl.kernel,
        out_type=jax.ShapeDtypeStruct((2, NPAD, DIM), jnp.float32),
        mesh=_sc_mesh(),
        scratch_types=[pltpu.VMEM((n_chunks, CHUNK), jnp.int32),
                       pltpu.VMEM((epw, DIM), jnp.float32),
                       pltpu.VMEM_SHARED((NPAD, DIM), jnp.float32),
                       pltpu.SemaphoreType.DMA],
        compiler_params=_SC_PARAMS,
    )
    def kern(m_hbm, idx_hbm, z_hbm, out_hbm, idx_v, m_v, acc, sem):
        cid = lax.axis_index("c")
        sid = lax.axis_index("s")
        wid = sid * 2 + cid
        slab = pl.ds(sid * RS, RS)
        pltpu.sync_copy(z_hbm.at[slab], acc.at[slab])
        d1 = pltpu.async_copy(idx_hbm.at[wid], idx_v, sem)
        d2 = pltpu.async_copy(m_hbm.at[pl.ds(wid * epw, epw)], m_v, sem)
        d1.wait()
        d2.wait()
        plsc.subcore_barrier()

        @pl.loop(0, n_chunks, step=4)
        def _(g):
            descs = [
                pltpu.async_copy(m_v.at[pl.ds((g + u) * CHUNK, CHUNK)],
                                 acc.at[idx_v.at[g + u]], sem, add=True)
                for u in range(4)
            ]
            for d in descs:
                d.wait()

        plsc.subcore_barrier()
        pltpu.sync_copy(acc.at[slab], out_hbm.at[cid, slab])

    return kern(m, dst3d, zeros_nd)


def _sc_count(dst3d, ones_cd, zeros_nd):
    @functools.partial(
        pl.kernel,
        out_type=jax.ShapeDtypeStruct((2, NPAD, DIM), jnp.float32),
        mesh=_sc_mesh(),
        scratch_types=[pltpu.VMEM((JG, CHUNK), jnp.int32),
                       pltpu.VMEM((CHUNK, DIM), jnp.float32),
                       pltpu.VMEM_SHARED((NPAD, DIM), jnp.float32),
                       pltpu.SemaphoreType.DMA],
        compiler_params=_SC_PARAMS,
    )
    def kern(idx_hbm, ones_hbm, z_hbm, out_hbm, idx_v, ones_v, acc, sem):
        cid = lax.axis_index("c")
        sid = lax.axis_index("s")
        wid = sid * 2 + cid
        slab = pl.ds(sid * RS, RS)
        pltpu.sync_copy(z_hbm.at[slab], acc.at[slab])
        pltpu.sync_copy(ones_hbm, ones_v)
        pltpu.sync_copy(idx_hbm.at[wid], idx_v)
        plsc.subcore_barrier()

        @pl.loop(0, JG, step=4)
        def _(g):
            descs = [
                pltpu.async_copy(ones_v, acc.at[idx_v.at[g + u]], sem,
                                 add=True)
                for u in range(4)
            ]
            for d in descs:
                d.wait()

        plsc.subcore_barrier()
        pltpu.sync_copy(acc.at[slab], out_hbm.at[cid, slab])

    return kern(dst3d, ones_cd, zeros_nd)


def kernel(x, edge_attr, edge_attr3, Wn, bn, We, be, W1, b1, W2, b2, conv_bias,
           w_ih, w_hh, b_ih, b_hh, Ww, Wb, gamma, beta, edge_index,
           edge_index3):
    f32 = jnp.float32
    xp = jnp.zeros((NPAD, 8), f32).at[:NN].set(x)
    attr = jnp.zeros((PAD_E, 19), f32).at[:NE].set(edge_attr)
    srcf = jnp.zeros((PAD_E,), jnp.int32).at[:NE].set(edge_index[0])
    dstf = jnp.full((PAD_E,), NN, jnp.int32).at[:NE].set(edge_index[1])
    HALF = PAD_E // 2
    JGH = JG // 2
    src_h = [srcf[:HALF].reshape(NW, JGH, CHUNK),
             srcf[HALF:].reshape(NW, JGH, CHUNK)]
    dst_h = [dstf[:HALF].reshape(NW, JGH, CHUNK),
             dstf[HALF:].reshape(NW, JGH, CHUNK)]
    dst3d = dstf.reshape(NW, JG, CHUNK)
    s3 = jnp.zeros((N3PAD,), jnp.int32).at[:NE3].set(edge_index3[0])
    d3 = jnp.zeros((N3PAD,), jnp.int32).at[:NE3].set(edge_index3[1])
    idx3 = jnp.concatenate([s3, d3]).reshape(NW, JG3, CHUNK)
    ea3 = jnp.zeros((N3PAD, 8), f32).at[:NE3].set(edge_attr3)
    zeros_nd = jnp.zeros((NPAD, DIM), f32)
    ones_cd = jnp.ones((CHUNK, DIM), f32)
    rt = jnp.repeat(jnp.eye(DIM, dtype=f32), DIM, axis=1)

    wnt = Wn.T
    wet = We.T
    w1t = W1.T
    w2t = W2.T
    wiht = w_ih.T
    whht = w_hh.T
    wwt = Ww.T
    wbt = Wb.T
    bn2 = bn.reshape(1, DIM)
    be2 = be.reshape(1, 12)
    b12 = b1.reshape(1, 128)
    b22 = b2.reshape(1, 256)
    cb2 = conv_bias.reshape(1, DIM)
    bih2 = b_ih.reshape(1, 48)
    bhh2 = b_hh.reshape(1, 48)
    g2 = gamma.reshape(1, 48)
    bt2 = beta.reshape(1, 48)

    out0 = pl.pallas_call(
        _k0_body,
        out_shape=jax.ShapeDtypeStruct((NPAD, DIM), f32),
    )(xp, wnt, bn2)

    n_eb = PAD_E // BE
    hh = pl.pallas_call(
        _k1_body,
        grid=(n_eb,),
        in_specs=[pl.BlockSpec((BE, 19), lambda i: (i, 0)),
                  pl.BlockSpec((19, 12), lambda i: (0, 0)),
                  pl.BlockSpec((1, 12), lambda i: (0, 0)),
                  pl.BlockSpec((12, 128), lambda i: (0, 0)),
                  pl.BlockSpec((1, 128), lambda i: (0, 0))],
        out_specs=pl.BlockSpec((BE, 128), lambda i: (i, 0)),
        out_shape=jax.ShapeDtypeStruct((PAD_E, 128), jnp.bfloat16),
    )(attr, wet, be2, w1t, b12)

    cnt = _sc_count(dst3d, ones_cd, zeros_nd)

    n_ebh = HALF // BE
    w2tb = w2t.astype(jnp.bfloat16)

    def km_half(xj, hf):
        return pl.pallas_call(
            _km_body,
            grid=(n_ebh,),
            in_specs=[pl.BlockSpec((BE, DIM), lambda i: (i, 0)),
                      pl.BlockSpec((BE, 128),
                                   lambda i, hf=hf: (i + hf * n_ebh, 0)),
                      pl.BlockSpec((128, 256), lambda i: (0, 0)),
                      pl.BlockSpec((1, 256), lambda i: (0, 0)),
                      pl.BlockSpec((DIM, 256), lambda i: (0, 0))],
            out_specs=pl.BlockSpec((BE, DIM), lambda i: (i, 0)),
            out_shape=jax.ShapeDtypeStruct((HALF, DIM), f32),
        )(xj, hh, w2tb, b22, rt)

    h = out0
    for _ in range(3):
        xj0 = _sc_gather(h, src_h[0], JGH, HALF, 10)
        m0 = km_half(xj0, 0)
        xj1 = _sc_gather(h, src_h[1], JGH, HALF, 10)
        m1 = km_half(xj1, 1)
        part0 = _sc_scatter_add(m0, dst_h[0], zeros_nd, JGH)
        part1 = _sc_scatter_add(m1, dst_h[1], zeros_nd, JGH)
        h = pl.pallas_call(
            _kgru_body,
            out_shape=jax.ShapeDtypeStruct((NPAD, DIM), f32),
        )(part0, part1, cnt, h, cb2, wiht, bih2, whht, bhh2)

    temp = _sc_gather(h, idx3, JG3, 2 * N3PAD, 7)

    n_rb = N3PAD // BR
    t0spec = pl.BlockSpec((BR, DIM), lambda i: (i, 0))
    t1spec = pl.BlockSpec((BR, DIM), lambda i: (i + n_rb, 0))
    stats = pl.pallas_call(
        _kr1_body,
        grid=(n_rb,),
        in_specs=[t0spec, t1spec],
        out_specs=pl.BlockSpec((2, 48), lambda i: (0, 0)),
        out_shape=jax.ShapeDtypeStruct((2, 48), f32),
    )(temp, temp)
    res = pl.pallas_call(
        _kr2_body,
        grid=(n_rb,),
        in_specs=[t0spec, t1spec,
                  pl.BlockSpec((BR, 8), lambda i: (i, 0)),
                  pl.BlockSpec((2, 48), lambda i: (0, 0)),
                  pl.BlockSpec((8, 48), lambda i: (0, 0)),
                  pl.BlockSpec((8, 1), lambda i: (0, 0)),
                  pl.BlockSpec((1, 48), lambda i: (0, 0)),
                  pl.BlockSpec((1, 48), lambda i: (0, 0))],
        out_specs=pl.BlockSpec((BR, 1), lambda i: (i, 0)),
        out_shape=jax.ShapeDtypeStruct((N3PAD, 1), f32),
    )(temp, temp, ea3, stats, wwt, wbt, g2, bt2)
    return res[:NE3, 0]
